# baseline (speedup 1.0000x reference)
import functools

import jax
import jax.numpy as jnp
from jax import lax
from jax.experimental import pallas as pl
from jax.experimental.pallas import tpu as pltpu
from jax.experimental.pallas import tpu_sc as plsc

_Q, _S, _C = 1000, 300, 256
_THR = 0.2
_SP = 304
_K = 4
_QP = 1024
_BQ = 8
_NCH = 1000 // _BQ
_NBUF = 6
_NDMA = 4
_NL = 16
_NSL = _SP // _NL
_SPV = _SP + _NL
_QKP = 4096


def _sim_body(pilot_ref, src_hbm, out_ref, bufs, sems):
    def start(i, slot):
        for d in range(_NDMA):
            w = _BQ // _NDMA
            pltpu.make_async_copy(
                src_hbm.at[pl.ds(i * _BQ + d * w, w)],
                bufs.at[slot, pl.ds(d * w, w)],
                sems.at[slot, d],
            ).start()

    def wait(slot):
        for d in range(_NDMA):
            w = _BQ // _NDMA
            pltpu.make_async_copy(
                src_hbm.at[pl.ds(0, w)],
                bufs.at[slot, pl.ds(d * w, w)],
                sems.at[slot, d],
            ).wait()

    for i in range(_NBUF):
        start(i, i)

    pt = pilot_ref[...].T

    ii = lax.broadcasted_iota(jnp.int32, (_BQ, _S, _S), 1)
    jj = lax.broadcasted_iota(jnp.int32, (_BQ, _S, _S), 2)
    eye = ii == jj

    def step(i, carry):
        slot = lax.rem(i, _NBUF)
        wait(slot)
        flat = bufs[slot].reshape(_BQ * _S, _C)
        full = lax.dot_general(flat, pt, (((1,), (0,)), ((), ())))
        full3 = full.reshape(_BQ, _S, _S)
        out_ref[pl.ds(i * _BQ, _BQ), :] = jnp.sum(
            jnp.where(eye, full3, 0.0), axis=-1)
        nxt = i + _NBUF

        @pl.when(nxt < _NCH)
        def _():
            start(nxt, slot)

        return carry

    lax.fori_loop(0, _NCH, step, jnp.int32(0))


def _scores_body(sim_ref, ms_ref, tkidx_ref, tkval_ref):
    sim = sim_ref[...]
    rmax = jnp.max(sim, axis=1, keepdims=True)
    rexp = jnp.exp(sim - rmax)
    d2t = rexp / jnp.sum(rexp, axis=1, keepdims=True)
    cmax = jnp.max(sim, axis=0, keepdims=True)
    cexp = jnp.exp(sim - cmax)
    t2d = cexp / jnp.sum(cexp, axis=0, keepdims=True)
    ms = (d2t + t2d) * 0.5
    ms_ref[...] = ms
    lane = lax.broadcasted_iota(jnp.int32, (_Q, _S), 1)
    work = ms
    for k in range(_K):
        m = jnp.max(work, axis=1, keepdims=True)
        cand = jnp.min(jnp.where(work == m, lane, _S), axis=1, keepdims=True)
        tkidx_ref[:, k:k + 1] = cand
        tkval_ref[:, k:k + 1] = m
        work = jnp.where(lane == cand, 0.0, work)


def _greedy_sc(ms_hbm, tkidx_hbm, tkval_hbm, out_hbm,
               tkidx_v, tkval_v, avail_v, ids_v, row_v, avail_s):
    first = (lax.axis_index("c") == 0) & (lax.axis_index("s") == 0)

    @pl.when(first)
    def _():
        pltpu.sync_copy(tkidx_hbm, tkidx_v)
        pltpu.sync_copy(tkval_hbm, tkval_v)
        for i in range(_SPV // _NL):
            avail_v[pl.ds(i * _NL, _NL)] = jnp.ones((_NL,), jnp.float32)
        lanes = lax.iota(jnp.int32, _NL)

        def fallback(q):
            pltpu.sync_copy(ms_hbm.at[q], row_v)
            m = jnp.full((_NL,), -1.0, jnp.float32)
            li = jnp.zeros((_NL,), jnp.int32)
            for i in range(_NSL):
                v = row_v[pl.ds(i * _NL, _NL)] * avail_v[pl.ds(i * _NL, _NL)]
                upd = v > m
                m = jnp.where(upd, v, m)
                li = jnp.where(upd, lanes + (i * _NL), li)
            gmax = m[0]
            for j in range(1, _NL):
                gmax = jnp.maximum(gmax, m[j])
            big = jnp.int32(2 ** 30)
            first_li = big
            for j in range(_NL):
                first_li = jnp.minimum(
                    first_li, jnp.where(m[j] == gmax, li[j], big))
            return jnp.where(gmax > _THR, first_li, jnp.int32(-1))

        rows_per = _NL // _K

        def step(b, carry):
            sentinel = jnp.int32(-2)
            iv = tkidx_v[pl.ds(b * _NL, _NL)]
            vv = tkval_v[pl.ds(b * _NL, _NL)]
            idvec = ids_v[pl.ds(b * rows_per, _NL)]
            for r in range(rows_per):
                best = sentinel
                for j in reversed(range(_K)):
                    cj = iv[r * _K + j]
                    vj = vv[r * _K + j]
                    free = avail_v[pl.ds(cj, _NL)][0] > 0.0
                    best = jnp.where(vj <= _THR, jnp.int32(-1),
                                     jnp.where(free, cj, best))
                idq = lax.cond(best == sentinel,
                               lambda rr=r: fallback(b * rows_per + rr),
                               lambda: best)
                idvec = jnp.where(lanes == r, idq, idvec)

                @pl.when(idq >= 0)
                def _take():
                    s = avail_v[pl.ds(idq, _NL)]
                    avail_v[pl.ds(idq, _NL)] = jnp.where(lanes == 0, 0.0, s)

            ids_v[pl.ds(b * rows_per, _NL)] = idvec
            return carry

        lax.fori_loop(0, _Q // rows_per, step, jnp.int32(0))
        pltpu.sync_copy(ids_v, out_hbm)


@jax.jit
def kernel(pilot_reid_embeds, src_reid_embeds):
    sim = pl.pallas_call(
        _sim_body,
        in_specs=[
            pl.BlockSpec((_S, _C), lambda: (0, 0)),
            pl.BlockSpec(memory_space=pl.ANY),
        ],
        out_specs=pl.BlockSpec((_Q, _S), lambda: (0, 0)),
        out_shape=jax.ShapeDtypeStruct((_Q, _S), jnp.float32),
        scratch_shapes=[
            pltpu.VMEM((_NBUF, _BQ, _S, _C), jnp.float32),
            pltpu.SemaphoreType.DMA((_NBUF, _NDMA)),
        ],
    )(pilot_reid_embeds, src_reid_embeds)

    ms, tkidx, tkval = pl.pallas_call(
        _scores_body,
        out_shape=(
            jax.ShapeDtypeStruct((_Q, _S), jnp.float32),
            jax.ShapeDtypeStruct((_Q, _K), jnp.int32),
            jax.ShapeDtypeStruct((_Q, _K), jnp.float32),
        ),
    )(sim)

    ms_p = jnp.pad(ms, ((0, 0), (0, _SP - _S)))
    tkidx_f = jnp.pad(tkidx.reshape(-1), (0, _QKP - _Q * _K))
    tkval_f = jnp.pad(tkval.reshape(-1), (0, _QKP - _Q * _K))

    greedy = pl.kernel(
        _greedy_sc,
        out_type=jax.ShapeDtypeStruct((_QP,), jnp.int32),
        mesh=plsc.VectorSubcoreMesh(core_axis_name="c", subcore_axis_name="s"),
        scratch_types=[
            pltpu.VMEM((_QKP,), jnp.int32),
            pltpu.VMEM((_QKP,), jnp.float32),
            pltpu.VMEM((_SPV,), jnp.float32),
            pltpu.VMEM((_QP,), jnp.int32),
            pltpu.VMEM((_SP,), jnp.float32),
            pltpu.SMEM((_SP,), jnp.int32),
        ],
    )
    ids = greedy(ms_p, tkidx_f, tkval_f)
    return ids[:_Q]


# speedup vs baseline: 1.1876x; 1.1876x over previous
import functools

import jax
import jax.numpy as jnp
from jax import lax
from jax.experimental import pallas as pl
from jax.experimental.pallas import tpu as pltpu
from jax.experimental.pallas import tpu_sc as plsc

_Q, _S, _C = 1000, 300, 256
_THR = 0.2
_SP = 304
_K = 4
_QP = 1024
_BQ = 8
_NCH = 1000 // _BQ
_NBUF = 6
_NDMA = 4
_NL = 16
_NSL = _SP // _NL
_SPV = _SP + _NL
_QKP = 4096


def _sim_body(pilot_ref, src_hbm, out_ref, bufs, sems):
    def start(i, slot):
        for d in range(_NDMA):
            w = _BQ // _NDMA
            pltpu.make_async_copy(
                src_hbm.at[pl.ds(i * _BQ + d * w, w)],
                bufs.at[slot, pl.ds(d * w, w)],
                sems.at[slot, d],
            ).start()

    def wait(slot):
        for d in range(_NDMA):
            w = _BQ // _NDMA
            pltpu.make_async_copy(
                src_hbm.at[pl.ds(0, w)],
                bufs.at[slot, pl.ds(d * w, w)],
                sems.at[slot, d],
            ).wait()

    for i in range(_NBUF):
        start(i, i)

    pt = pilot_ref[...].T

    ii = lax.broadcasted_iota(jnp.int32, (_BQ, _S, _S), 1)
    jj = lax.broadcasted_iota(jnp.int32, (_BQ, _S, _S), 2)
    eye = ii == jj

    def step(i, carry):
        slot = lax.rem(i, _NBUF)
        wait(slot)
        flat = bufs[slot].reshape(_BQ * _S, _C)
        full = lax.dot_general(flat, pt, (((1,), (0,)), ((), ())))
        full3 = full.reshape(_BQ, _S, _S)
        out_ref[pl.ds(i * _BQ, _BQ), :] = jnp.sum(
            jnp.where(eye, full3, 0.0), axis=-1)
        nxt = i + _NBUF

        @pl.when(nxt < _NCH)
        def _():
            start(nxt, slot)

        return carry

    lax.fori_loop(0, _NCH, step, jnp.int32(0))


def _scores_body(sim_ref, ms_ref, tkidx_ref, tkval_ref):
    sim = sim_ref[...]
    rmax = jnp.max(sim, axis=1, keepdims=True)
    rexp = jnp.exp(sim - rmax)
    d2t = rexp / jnp.sum(rexp, axis=1, keepdims=True)
    cmax = jnp.max(sim, axis=0, keepdims=True)
    cexp = jnp.exp(sim - cmax)
    t2d = cexp / jnp.sum(cexp, axis=0, keepdims=True)
    ms = (d2t + t2d) * 0.5
    ms_ref[...] = ms
    lane = lax.broadcasted_iota(jnp.int32, (_Q, _S), 1)
    work = ms
    for k in range(_K):
        m = jnp.max(work, axis=1, keepdims=True)
        cand = jnp.min(jnp.where(work == m, lane, _S), axis=1, keepdims=True)
        tkidx_ref[:, k:k + 1] = cand
        tkval_ref[:, k:k + 1] = m
        work = jnp.where(lane == cand, 0.0, work)


def _greedy_sc(ms_hbm, tkidx_hbm, tkval_hbm, out_hbm,
               tkidx_v, tkval_v, avail_v, ids_v, row_v, avail_s):
    first = (lax.axis_index("c") == 0) & (lax.axis_index("s") == 0)

    @pl.when(first)
    def _():
        pltpu.sync_copy(tkidx_hbm, tkidx_v)
        pltpu.sync_copy(tkval_hbm, tkval_v)
        for i in range(_SPV // _NL):
            avail_v[pl.ds(i * _NL, _NL)] = jnp.ones((_NL,), jnp.float32)
        lanes = lax.iota(jnp.int32, _NL)

        def fallback(q):
            pltpu.sync_copy(ms_hbm.at[q], row_v)
            m = jnp.full((_NL,), -1.0, jnp.float32)
            li = jnp.zeros((_NL,), jnp.int32)
            for i in range(_NSL):
                v = row_v[pl.ds(i * _NL, _NL)] * avail_v[pl.ds(i * _NL, _NL)]
                upd = v > m
                m = jnp.where(upd, v, m)
                li = jnp.where(upd, lanes + (i * _NL), li)
            gmax = m[0]
            for j in range(1, _NL):
                gmax = jnp.maximum(gmax, m[j])
            big = jnp.int32(2 ** 30)
            first_li = big
            for j in range(_NL):
                first_li = jnp.minimum(
                    first_li, jnp.where(m[j] == gmax, li[j], big))
            return jnp.where(gmax > _THR, first_li, jnp.int32(-1))

        rows_per = _NL // _K

        def mark_taken(i):
            s = avail_v[pl.ds(i, _NL)]
            avail_v[pl.ds(i, _NL)] = jnp.where(lanes == 0, 0.0, s)

        def fallback_flush(q, earlier):
            for e in earlier:
                @pl.when(e >= 0)
                def _f(i=e):
                    mark_taken(i)
            return fallback(q)

        def step(b, carry):
            sentinel = jnp.int32(-2)
            iv = tkidx_v[pl.ds(b * _NL, _NL)]
            vv = tkval_v[pl.ds(b * _NL, _NL)]
            idvec = ids_v[pl.ds(b * rows_per, _NL)]
            frees = [avail_v[pl.ds(iv[t], _NL)][0] > 0.0 for t in range(_NL)]
            taken_ids = []
            for r in range(rows_per):
                best = sentinel
                for j in reversed(range(_K)):
                    t = r * _K + j
                    cj = iv[t]
                    free = frees[t]
                    for tid in taken_ids:
                        free = free & (cj != tid)
                    best = jnp.where(vv[t] <= _THR, jnp.int32(-1),
                                     jnp.where(free, cj, best))
                idq = lax.cond(
                    best == sentinel,
                    lambda rr=r, tk=tuple(taken_ids): fallback_flush(
                        b * rows_per + rr, tk),
                    lambda: best)
                idvec = jnp.where(lanes == r, idq, idvec)
                taken_ids.append(idq)

            ids_v[pl.ds(b * rows_per, _NL)] = idvec
            for idq in taken_ids:
                @pl.when(idq >= 0)
                def _take(i=idq):
                    mark_taken(i)
            return carry

        lax.fori_loop(0, _Q // rows_per, step, jnp.int32(0))
        pltpu.sync_copy(ids_v, out_hbm)


@jax.jit
def kernel(pilot_reid_embeds, src_reid_embeds):
    sim = pl.pallas_call(
        _sim_body,
        in_specs=[
            pl.BlockSpec((_S, _C), lambda: (0, 0)),
            pl.BlockSpec(memory_space=pl.ANY),
        ],
        out_specs=pl.BlockSpec((_Q, _S), lambda: (0, 0)),
        out_shape=jax.ShapeDtypeStruct((_Q, _S), jnp.float32),
        scratch_shapes=[
            pltpu.VMEM((_NBUF, _BQ, _S, _C), jnp.float32),
            pltpu.SemaphoreType.DMA((_NBUF, _NDMA)),
        ],
    )(pilot_reid_embeds, src_reid_embeds)

    ms, tkidx, tkval = pl.pallas_call(
        _scores_body,
        out_shape=(
            jax.ShapeDtypeStruct((_Q, _S), jnp.float32),
            jax.ShapeDtypeStruct((_Q, _K), jnp.int32),
            jax.ShapeDtypeStruct((_Q, _K), jnp.float32),
        ),
    )(sim)

    ms_p = jnp.pad(ms, ((0, 0), (0, _SP - _S)))
    tkidx_f = jnp.pad(tkidx.reshape(-1), (0, _QKP - _Q * _K))
    tkval_f = jnp.pad(tkval.reshape(-1), (0, _QKP - _Q * _K))

    greedy = pl.kernel(
        _greedy_sc,
        out_type=jax.ShapeDtypeStruct((_QP,), jnp.int32),
        mesh=plsc.VectorSubcoreMesh(core_axis_name="c", subcore_axis_name="s"),
        scratch_types=[
            pltpu.VMEM((_QKP,), jnp.int32),
            pltpu.VMEM((_QKP,), jnp.float32),
            pltpu.VMEM((_SPV,), jnp.float32),
            pltpu.VMEM((_QP,), jnp.int32),
            pltpu.VMEM((_SP,), jnp.float32),
            pltpu.SMEM((_SP,), jnp.int32),
        ],
    )
    ids = greedy(ms_p, tkidx_f, tkval_f)
    return ids[:_Q]
